# baseline (speedup 1.0000x reference)
import jax
import jax.numpy as jnp
from jax import lax
from jax.experimental import pallas as pl
from jax.experimental.pallas import tpu as pltpu
from jax.experimental.pallas import tpu_sc as plsc

_MAP = 64
_K = 64
_B = 64
_C = 145
_S2 = _MAP * _MAP
_THR = 0.25

_NC = 2
_NS = 16
_NW = _NC * _NS
_DETS = _B * _K
_DPT = _DETS // _NW
_EPT = _DPT * _C


def _shift2d(x, dy, dx, fill):
    b, h, w = x.shape
    if dy > 0:
        x = jnp.concatenate([x[:, dy:, :], jnp.full((b, dy, w), fill, x.dtype)], axis=1)
    elif dy < 0:
        x = jnp.concatenate([jnp.full((b, -dy, w), fill, x.dtype), x[:, :dy, :]], axis=1)
    if dx > 0:
        x = jnp.concatenate([x[:, :, dx:], jnp.full((b, h, dx), fill, x.dtype)], axis=2)
    elif dx < 0:
        x = jnp.concatenate([jnp.full((b, h, -dx), fill, x.dtype), x[:, :, :dx]], axis=2)
    return x


def _nms_topk_kernel(cm_ref, score_ref, ind_ref, cy_ref, cx_ref, valid_ref,
                     idx_all_ref):
    cm = cm_ref[...]
    neg = jnp.float32(-jnp.inf)
    pooled = cm
    for dy in (-1, 0, 1):
        for dx in (-1, 0, 1):
            if dy == 0 and dx == 0:
                continue
            pooled = jnp.maximum(pooled, _shift2d(cm, dy, dx, neg))
    vals = jnp.where(pooled == cm, cm, jnp.float32(0.0))

    flatidx = (lax.broadcasted_iota(jnp.int32, cm.shape, 1) * _MAP
               + lax.broadcasted_iota(jnp.int32, cm.shape, 2))
    kcol = lax.broadcasted_iota(jnp.int32, (_B, _K), 1)

    def body(k, carry):
        vals, scores, inds = carry
        m = jnp.max(vals, axis=(1, 2))
        cand = jnp.where(vals == m[:, None, None], flatidx, jnp.int32(_S2))
        idx = jnp.min(cand, axis=(1, 2))
        vals = jnp.where(flatidx == idx[:, None, None], neg, vals)
        scores = jnp.where(kcol == k, m[:, None], scores)
        inds = jnp.where(kcol == k, idx[:, None], inds)
        return vals, scores, inds

    scores0 = jnp.zeros((_B, _K), jnp.float32)
    inds0 = jnp.zeros((_B, _K), jnp.int32)
    _, scores, inds = lax.fori_loop(0, _K, body, (vals, scores0, inds0))

    score_ref[...] = scores
    ind_ref[...] = inds
    cy_ref[...] = inds // _MAP
    cx_ref[...] = inds % _MAP
    valid_ref[...] = scores > _THR

    b_off = lax.broadcasted_iota(jnp.int32, (_B, _K, _C), 0) * (_C * _S2)
    c_off = lax.broadcasted_iota(jnp.int32, (_B, _K, _C), 2) * _S2
    idx_all_ref[...] = b_off + c_off + inds[:, :, None]


def _sc_gather_kernel(table_hbm, idx_hbm, out_hbm, idx_v, rows_v, sem):
    wid = lax.axis_index("s") * _NC + lax.axis_index("c")
    base = wid * _EPT
    pltpu.sync_copy(idx_hbm.at[pl.ds(base, _EPT)], idx_v)
    pltpu.async_copy(table_hbm.at[idx_v], rows_v, sem).wait()
    pltpu.sync_copy(rows_v, out_hbm.at[pl.ds(base, _EPT)])


def _sc_gather(params_maps, idx_all):
    table = params_maps.reshape(_B * _C * _S2)
    mesh = plsc.VectorSubcoreMesh(core_axis_name="c", subcore_axis_name="s")
    f = pl.kernel(
        _sc_gather_kernel,
        out_type=jax.ShapeDtypeStruct((_DETS * _C,), jnp.float32),
        mesh=mesh,
        scratch_types=[
            pltpu.VMEM((_EPT,), jnp.int32),
            pltpu.VMEM((_EPT,), jnp.float32),
            pltpu.SemaphoreType.DMA,
        ],
    )
    return f(table, idx_all.reshape(_DETS * _C))


def kernel(center_map, params_maps):
    cm = center_map[:, 0]

    scores, inds, cy, cx, valid, idx_all = pl.pallas_call(
        _nms_topk_kernel,
        out_shape=(
            jax.ShapeDtypeStruct((_B, _K), jnp.float32),
            jax.ShapeDtypeStruct((_B, _K), jnp.int32),
            jax.ShapeDtypeStruct((_B, _K), jnp.int32),
            jax.ShapeDtypeStruct((_B, _K), jnp.int32),
            jax.ShapeDtypeStruct((_B, _K), jnp.bool_),
            jax.ShapeDtypeStruct((_B, _K, _C), jnp.int32),
        ),
    )(cm)

    gathered = _sc_gather(params_maps, idx_all)
    params_pred = gathered.reshape(_DETS, _C)

    cyxs = jnp.stack([cy, cx], axis=-1)
    reorganize_idx = jnp.repeat(jnp.arange(_B, dtype=jnp.int32), _K)
    return (params_pred, scores, valid, cyxs, reorganize_idx)
